# baseline (speedup 1.0000x reference)
import functools

import jax
import jax.numpy as jnp
from jax import lax
from jax.experimental import pallas as pl
from jax.experimental.pallas import tpu as pltpu
from jax.experimental.pallas import tpu_sc as plsc

N = 10000
D = 128
H = 16
C = 64

NP = 10240
TILES = 32
CHUNK = 128
ROWS_PER_TILE = NP // TILES
BM = 1024


def _sc_mesh():
    return plsc.VectorSubcoreMesh(core_axis_name="c", subcore_axis_name="s")


_SC_PARAMS = pltpu.CompilerParams(use_tc_tiling_on_sc=False)


def _make_deg_kernel(n_chunks):
    @functools.partial(
        pl.kernel,
        mesh=_sc_mesh(),
        out_type=jax.ShapeDtypeStruct((2, NP, H), jnp.float32),
        scratch_types=[
            pltpu.VMEM((n_chunks, CHUNK), jnp.int32),
            pltpu.VMEM((CHUNK, H), jnp.float32),
            pltpu.SemaphoreType.DMA,
            pltpu.VMEM_SHARED((NP, H), jnp.float32),
        ],
        compiler_params=_SC_PARAMS,
    )
    def deg_kernel(dst_hbm, zeros_hbm, ones_hbm, out_hbm, idx_v, ones_v,
                   ssem, acc_sh):
        c = lax.axis_index("c")
        s = lax.axis_index("s")
        tg = c * 16 + s
        row0 = s * ROWS_PER_TILE
        pltpu.sync_copy(dst_hbm.at[tg], idx_v)
        pltpu.sync_copy(ones_hbm, ones_v)
        pltpu.sync_copy(zeros_hbm, acc_sh.at[pl.ds(row0, ROWS_PER_TILE)])
        plsc.subcore_barrier()

        @pl.loop(0, n_chunks)
        def _(j):
            pltpu.sync_copy(ones_v, acc_sh.at[idx_v.at[j]], add=True)

        plsc.subcore_barrier()
        pltpu.sync_copy(
            acc_sh.at[pl.ds(row0, ROWS_PER_TILE)],
            out_hbm.at[c].at[pl.ds(row0, ROWS_PER_TILE)],
        )

    return deg_kernel


def _make_msg_kernel(n_chunks):
    @functools.partial(
        pl.kernel,
        mesh=_sc_mesh(),
        out_type=jax.ShapeDtypeStruct((2, NP, H), jnp.float32),
        scratch_types=[
            pltpu.VMEM((n_chunks, CHUNK), jnp.int32),
            pltpu.VMEM((n_chunks, CHUNK), jnp.int32),
            pltpu.VMEM((CHUNK, H), jnp.float32),
            pltpu.VMEM((CHUNK, H), jnp.float32),
            pltpu.SemaphoreType.DMA,
            pltpu.SemaphoreType.DMA,
            pltpu.VMEM_SHARED((NP, H), jnp.float32),
        ],
        compiler_params=_SC_PARAMS,
    )
    def msg_kernel(table_hbm, src_hbm, dst_hbm, zeros_hbm, out_hbm,
                   src_v, dst_v, r0, r1, s0, s1, acc_sh):
        c = lax.axis_index("c")
        s = lax.axis_index("s")
        tg = c * 16 + s
        row0 = s * ROWS_PER_TILE
        pltpu.sync_copy(src_hbm.at[tg], src_v)
        pltpu.sync_copy(dst_hbm.at[tg], dst_v)
        pltpu.sync_copy(zeros_hbm, acc_sh.at[pl.ds(row0, ROWS_PER_TILE)])
        plsc.subcore_barrier()

        rows = (r0, r1)
        sems = (s0, s1)
        gcp = pltpu.async_copy(table_hbm.at[src_v.at[0]], rows[0], sems[0])
        for j in range(n_chunks):
            nxt = None
            if j + 1 < n_chunks:
                b = (j + 1) % 2
                nxt = pltpu.async_copy(table_hbm.at[src_v.at[j + 1]],
                                       rows[b], sems[b])
            gcp.wait()
            pltpu.sync_copy(rows[j % 2], acc_sh.at[dst_v.at[j]], add=True)
            gcp = nxt

        plsc.subcore_barrier()
        pltpu.sync_copy(
            acc_sh.at[pl.ds(row0, ROWS_PER_TILE)],
            out_hbm.at[c].at[pl.ds(row0, ROWS_PER_TILE)],
        )

    return msg_kernel


def _mm1_body(x_ref, w_ref, o_ref):
    o_ref[...] = jnp.dot(x_ref[...], w_ref[...],
                         preferred_element_type=jnp.float32)


def _scale_body(dp_ref, h_ref, o_ref):
    dis = lax.rsqrt(dp_ref[0] + dp_ref[1] + 1.0)
    o_ref[...] = dis * h_ref[...]


def _layer_body(dp_ref, p_ref, hp_ref, w_ref, b_ref, o_ref):
    dis = lax.rsqrt(dp_ref[0] + dp_ref[1] + 1.0)
    z = dis * (p_ref[0] + p_ref[1] + hp_ref[...]) + b_ref[...]
    a = jnp.maximum(z, 0.0)
    h2 = jnp.dot(a, w_ref[...], preferred_element_type=jnp.float32)
    o_ref[...] = dis * h2


def _out_body(dp_ref, q_ref, hp_ref, w_ref, b_ref, bo_ref, o_ref):
    dis = lax.rsqrt(dp_ref[0] + dp_ref[1] + 1.0)
    z = dis * (q_ref[0] + q_ref[1] + hp_ref[...]) + b_ref[...]
    o_ref[...] = jnp.dot(z, w_ref[...],
                         preferred_element_type=jnp.float32) + bo_ref[...]


def _blk(shape, imap):
    return pl.BlockSpec(shape, imap)


def kernel(x, edge_index, W1, b1, W2, b2, Wout, bout):
    E = edge_index.shape[1]
    cap = TILES * CHUNK
    n_chunks = 8 * (-(-E // (8 * cap)))
    e_pad = n_chunks * cap - E
    src = edge_index[0].astype(jnp.int32)
    dst = edge_index[1].astype(jnp.int32)
    if e_pad:
        fill = jnp.arange(e_pad, dtype=jnp.int32)
        src = jnp.concatenate([src, fill % N])
        dst = jnp.concatenate([dst, N + fill % (NP - N)])
    src_t = src.reshape(TILES, n_chunks, CHUNK)
    dst_t = dst.reshape(TILES, n_chunks, CHUNK)
    x_pad = jnp.pad(x, ((0, NP - N), (0, 0)))
    zeros_hbm = jnp.zeros((ROWS_PER_TILE, H), jnp.float32)
    ones_hbm = jnp.ones((CHUNK, H), jnp.float32)
    b1r = b1.reshape(1, H)
    b2r = b2.reshape(1, H)
    boutr = bout.reshape(1, C)

    grid = (NP // BM,)

    h1 = pl.pallas_call(
        _mm1_body,
        grid=grid,
        in_specs=[_blk((BM, D), lambda i: (i, 0)),
                  _blk((D, H), lambda i: (0, 0))],
        out_specs=_blk((BM, H), lambda i: (i, 0)),
        out_shape=jax.ShapeDtypeStruct((NP, H), jnp.float32),
    )(x_pad, W1)

    degp = _make_deg_kernel(n_chunks)(dst_t, zeros_hbm, ones_hbm)

    h1p = pl.pallas_call(
        _scale_body,
        grid=grid,
        in_specs=[_blk((2, BM, H), lambda i: (0, i, 0)),
                  _blk((BM, H), lambda i: (i, 0))],
        out_specs=_blk((BM, H), lambda i: (i, 0)),
        out_shape=jax.ShapeDtypeStruct((NP, H), jnp.float32),
    )(degp, h1)

    p = _make_msg_kernel(n_chunks)(h1p, src_t, dst_t, zeros_hbm)

    h2p = pl.pallas_call(
        _layer_body,
        grid=grid,
        in_specs=[_blk((2, BM, H), lambda i: (0, i, 0)),
                  _blk((2, BM, H), lambda i: (0, i, 0)),
                  _blk((BM, H), lambda i: (i, 0)),
                  _blk((H, H), lambda i: (0, 0)),
                  _blk((1, H), lambda i: (0, 0))],
        out_specs=_blk((BM, H), lambda i: (i, 0)),
        out_shape=jax.ShapeDtypeStruct((NP, H), jnp.float32),
    )(degp, p, h1p, W2, b1r)

    q = _make_msg_kernel(n_chunks)(h2p, src_t, dst_t, zeros_hbm)

    out = pl.pallas_call(
        _out_body,
        grid=grid,
        in_specs=[_blk((2, BM, H), lambda i: (0, i, 0)),
                  _blk((2, BM, H), lambda i: (0, i, 0)),
                  _blk((BM, H), lambda i: (i, 0)),
                  _blk((H, C), lambda i: (0, 0)),
                  _blk((1, H), lambda i: (0, 0)),
                  _blk((1, C), lambda i: (0, 0))],
        out_specs=_blk((BM, C), lambda i: (i, 0)),
        out_shape=jax.ShapeDtypeStruct((NP, C), jnp.float32),
    )(degp, q, h2p, Wout, b2r, boutr)

    return out[:N]


# speedup vs baseline: 1.4147x; 1.4147x over previous
import functools

import jax
import jax.numpy as jnp
from jax import lax
from jax.experimental import pallas as pl
from jax.experimental.pallas import tpu as pltpu
from jax.experimental.pallas import tpu_sc as plsc

N = 10000
D = 128
H = 16
C = 64

NP = 10240
TILES = 32
CHUNK = 128
ROWS_PER_TILE = NP // TILES
NF = N * H // 128


def _sc_mesh():
    return plsc.VectorSubcoreMesh(core_axis_name="c", subcore_axis_name="s")


_SC_PARAMS = pltpu.CompilerParams(use_tc_tiling_on_sc=False)


def _tile_span(total_chunks):
    cmax = -(-total_chunks // TILES)

    def span(tg):
        lo = tg * total_chunks // TILES
        hi = (tg + 1) * total_chunks // TILES
        return lo, hi

    return cmax, span


def _make_deg_kernel(total_chunks):
    cmax, span = _tile_span(total_chunks)

    @functools.partial(
        pl.kernel,
        mesh=_sc_mesh(),
        out_type=jax.ShapeDtypeStruct((2, NP, H), jnp.float32),
        scratch_types=[
            pltpu.VMEM((cmax, 2, CHUNK), jnp.int32),
            pltpu.VMEM((CHUNK, H), jnp.float32),
            pltpu.VMEM_SHARED((NP, H), jnp.float32),
        ],
        compiler_params=_SC_PARAMS,
    )
    def deg_kernel(idx_hbm, zeros_hbm, ones_hbm, out_hbm, idx_v, ones_v,
                   acc_sh):
        c = lax.axis_index("c")
        s = lax.axis_index("s")
        tg = c * 16 + s
        lo, hi = span(tg)
        n_t = hi - lo
        row0 = s * ROWS_PER_TILE
        pltpu.sync_copy(idx_hbm.at[pl.ds(lo, cmax)], idx_v)
        pltpu.sync_copy(ones_hbm, ones_v)
        pltpu.sync_copy(zeros_hbm, acc_sh.at[pl.ds(row0, ROWS_PER_TILE)])
        plsc.subcore_barrier()

        for k in range(cmax - 1):
            pltpu.sync_copy(ones_v, acc_sh.at[idx_v.at[k].at[1]], add=True)

        @pl.when(cmax - 1 < n_t)
        def _():
            pltpu.sync_copy(ones_v, acc_sh.at[idx_v.at[cmax - 1].at[1]],
                            add=True)

        plsc.subcore_barrier()
        pltpu.sync_copy(
            acc_sh.at[pl.ds(row0, ROWS_PER_TILE)],
            out_hbm.at[c].at[pl.ds(row0, ROWS_PER_TILE)],
        )

    return deg_kernel


def _make_msg_kernel(total_chunks):
    cmax, span = _tile_span(total_chunks)

    @functools.partial(
        pl.kernel,
        mesh=_sc_mesh(),
        out_type=jax.ShapeDtypeStruct((2, NP, H), jnp.float32),
        scratch_types=[
            pltpu.VMEM((cmax, 2, CHUNK), jnp.int32),
            pltpu.VMEM((CHUNK, H), jnp.float32),
            pltpu.VMEM((CHUNK, H), jnp.float32),
            pltpu.SemaphoreType.DMA,
            pltpu.SemaphoreType.DMA,
            pltpu.VMEM_SHARED((NP, H), jnp.float32),
        ],
        compiler_params=_SC_PARAMS,
    )
    def msg_kernel(table_hbm, idx_hbm, zeros_hbm, out_hbm,
                   idx_v, r0, r1, s0, s1, acc_sh):
        c = lax.axis_index("c")
        s = lax.axis_index("s")
        tg = c * 16 + s
        lo, hi = span(tg)
        n_t = hi - lo
        row0 = s * ROWS_PER_TILE
        pltpu.sync_copy(idx_hbm.at[pl.ds(lo, cmax)], idx_v)
        pltpu.sync_copy(zeros_hbm, acc_sh.at[pl.ds(row0, ROWS_PER_TILE)])
        plsc.subcore_barrier()

        rows = (r0, r1)
        sems = (s0, s1)
        gcp = pltpu.async_copy(table_hbm.at[idx_v.at[0].at[0]], rows[0],
                               sems[0])
        for k in range(cmax - 1):
            b = (k + 1) % 2
            nxt = pltpu.async_copy(table_hbm.at[idx_v.at[k + 1].at[0]],
                                   rows[b], sems[b])
            gcp.wait()
            pltpu.sync_copy(rows[k % 2], acc_sh.at[idx_v.at[k].at[1]],
                            add=True)
            gcp = nxt
        gcp.wait()

        @pl.when(cmax - 1 < n_t)
        def _():
            pltpu.sync_copy(rows[(cmax - 1) % 2],
                            acc_sh.at[idx_v.at[cmax - 1].at[1]], add=True)

        plsc.subcore_barrier()
        pltpu.sync_copy(
            acc_sh.at[pl.ds(row0, ROWS_PER_TILE)],
            out_hbm.at[c].at[pl.ds(row0, ROWS_PER_TILE)],
        )

    return msg_kernel


def _mm1_body(x_ref, w_ref, o_ref):
    h = jnp.dot(x_ref[...], w_ref[...], preferred_element_type=jnp.float32)
    pad = o_ref.shape[0] - h.shape[0]
    o_ref[...] = jnp.concatenate(
        [h, jnp.zeros((pad, h.shape[1]), jnp.float32)], axis=0)


def _scale_body(dp_ref, h_ref, o_ref):
    dis = lax.rsqrt(dp_ref[0] + dp_ref[1] + 1.0)
    o_ref[...] = dis * h_ref[...]


def _layer_body(dp_ref, p_ref, hp_ref, w_ref, b_ref, o_ref):
    dis = lax.rsqrt(dp_ref[0] + dp_ref[1] + 1.0)
    z = dis * (p_ref[0] + p_ref[1] + hp_ref[...]) + b_ref[...]
    a = jnp.maximum(z, 0.0)
    h2 = jnp.dot(a, w_ref[...], preferred_element_type=jnp.float32)
    o_ref[...] = dis * h2


def _out_body(dp_ref, q_ref, hp_ref, w_ref, b_ref, bo_ref, o_ref):
    dis = lax.rsqrt(dp_ref[0] + dp_ref[1] + 1.0)
    z = dis * (q_ref[0] + q_ref[1] + hp_ref[...]) + b_ref[...]
    o_ref[...] = jnp.dot(z, w_ref[...],
                         preferred_element_type=jnp.float32) + bo_ref[...]


def _full(shape):
    return pl.BlockSpec(shape, lambda: tuple(0 for _ in shape))


def _tc_call(body, in_shapes, out_shape):
    return pl.pallas_call(
        body,
        in_specs=[_full(s) for s in in_shapes],
        out_specs=_full(out_shape),
        out_shape=jax.ShapeDtypeStruct(out_shape, jnp.float32),
    )


def kernel(x, edge_index, W1, b1, W2, b2, Wout, bout):
    E = edge_index.shape[1]
    total_chunks = E // CHUNK
    assert total_chunks * CHUNK == E

    idx3 = edge_index.astype(jnp.int32).reshape(2, total_chunks,
                                                CHUNK).transpose(1, 0, 2)
    x_flat = x.reshape(NF, 8 * D)
    eye8 = jnp.eye(8, dtype=jnp.float32)
    W1b = jnp.kron(eye8, W1)
    W2b = jnp.kron(eye8, W2)
    Woutb = jnp.kron(eye8, Wout)
    b1f = jnp.tile(b1, 8)[None]
    b2f = jnp.tile(b2, 8)[None]
    boutf = jnp.tile(bout, 8)[None]
    zeros_hbm = jnp.zeros((ROWS_PER_TILE, H), jnp.float32)
    ones_hbm = jnp.ones((CHUNK, H), jnp.float32)

    NPF = NP * H // 128

    h1f = _tc_call(_mm1_body, [(NF, 8 * D), (8 * D, D)], (NPF, D))(x_flat,
                                                                   W1b)

    degp = _make_deg_kernel(total_chunks)(idx3, zeros_hbm, ones_hbm)
    degpf = degp.reshape(2, NPF, 128)

    h1pf = _tc_call(_scale_body, [(2, NPF, 128), (NPF, 128)],
                    (NPF, 128))(degpf, h1f)

    p = _make_msg_kernel(total_chunks)(h1pf.reshape(NP, H), idx3, zeros_hbm)

    h2pf = _tc_call(
        _layer_body,
        [(2, NPF, 128), (2, NPF, 128), (NPF, 128), (128, 128), (1, 128)],
        (NPF, 128))(degpf, p.reshape(2, NPF, 128), h1pf, W2b, b1f)

    q = _make_msg_kernel(total_chunks)(h2pf.reshape(NP, H), idx3, zeros_hbm)

    outf = _tc_call(
        _out_body,
        [(2, NPF, 128), (2, NPF, 128), (NPF, 128), (128, 8 * C), (1, 128),
         (1, 8 * C)],
        (NPF, 8 * C))(degpf, q.reshape(2, NPF, 128), h2pf, Woutb, b2f, boutf)

    return outf.reshape(NP, C)[:N]


# speedup vs baseline: 1.4474x; 1.0231x over previous
import functools

import jax
import jax.numpy as jnp
from jax import lax
from jax.experimental import pallas as pl
from jax.experimental.pallas import tpu as pltpu
from jax.experimental.pallas import tpu_sc as plsc

N = 10000
D = 128
H = 16
C = 64

NP = 10240
TILES = 32
CHUNK = 128
ROWS_PER_TILE = NP // TILES
NF = N * H // 128


def _sc_mesh():
    return plsc.VectorSubcoreMesh(core_axis_name="c", subcore_axis_name="s")


_SC_PARAMS = pltpu.CompilerParams(use_tc_tiling_on_sc=False)


def _tile_span(total_chunks):
    cmax = -(-total_chunks // TILES)

    def span(tg):
        lo = tg * total_chunks // TILES
        hi = (tg + 1) * total_chunks // TILES
        return lo, hi

    return cmax, span


def _make_deg_kernel(total_chunks):
    cmax, span = _tile_span(total_chunks)

    @functools.partial(
        pl.kernel,
        mesh=_sc_mesh(),
        out_type=jax.ShapeDtypeStruct((2, NP, H), jnp.float32),
        scratch_types=[
            pltpu.VMEM((cmax, 2, CHUNK), jnp.int32),
            pltpu.VMEM((CHUNK, H), jnp.float32),
            pltpu.VMEM_SHARED((NP, H), jnp.float32),
        ],
        compiler_params=_SC_PARAMS,
    )
    def deg_kernel(idx_hbm, zeros_hbm, ones_hbm, out_hbm, idx_v, ones_v,
                   acc_sh):
        c = lax.axis_index("c")
        s = lax.axis_index("s")
        tg = c * 16 + s
        lo, hi = span(tg)
        n_t = hi - lo
        row0 = s * ROWS_PER_TILE
        pltpu.sync_copy(idx_hbm.at[pl.ds(lo, cmax)], idx_v)
        pltpu.sync_copy(ones_hbm, ones_v)
        pltpu.sync_copy(zeros_hbm, acc_sh.at[pl.ds(row0, ROWS_PER_TILE)])
        plsc.subcore_barrier()

        for k in range(cmax - 1):
            pltpu.sync_copy(ones_v, acc_sh.at[idx_v.at[k].at[1]], add=True)

        @pl.when(cmax - 1 < n_t)
        def _():
            pltpu.sync_copy(ones_v, acc_sh.at[idx_v.at[cmax - 1].at[1]],
                            add=True)

        plsc.subcore_barrier()
        pltpu.sync_copy(
            acc_sh.at[pl.ds(row0, ROWS_PER_TILE)],
            out_hbm.at[c].at[pl.ds(row0, ROWS_PER_TILE)],
        )

    return deg_kernel


def _make_msg_kernel(total_chunks):
    cmax, span = _tile_span(total_chunks)

    @functools.partial(
        pl.kernel,
        mesh=_sc_mesh(),
        out_type=jax.ShapeDtypeStruct((2, NP, H), jnp.float32),
        scratch_types=[
            pltpu.VMEM((cmax, 2, CHUNK), jnp.int32),
            pltpu.VMEM((CHUNK, H), jnp.float32),
            pltpu.VMEM((CHUNK, H), jnp.float32),
            pltpu.SemaphoreType.DMA,
            pltpu.SemaphoreType.DMA,
            pltpu.VMEM_SHARED((NP, H), jnp.float32),
        ],
        compiler_params=_SC_PARAMS,
    )
    def msg_kernel(table_hbm, idx_hbm, zeros_hbm, out_hbm,
                   idx_v, r0, r1, s0, s1, acc_sh):
        c = lax.axis_index("c")
        s = lax.axis_index("s")
        tg = c * 16 + s
        lo, hi = span(tg)
        n_t = hi - lo
        row0 = s * ROWS_PER_TILE
        pltpu.sync_copy(idx_hbm.at[pl.ds(lo, cmax)], idx_v)
        pltpu.sync_copy(zeros_hbm, acc_sh.at[pl.ds(row0, ROWS_PER_TILE)])
        plsc.subcore_barrier()

        rows = (r0, r1)
        gsems = (s0, s1)
        gcp = pltpu.async_copy(table_hbm.at[idx_v.at[0].at[0]], rows[0],
                               gsems[0])
        for k in range(cmax - 1):
            b = (k + 1) % 2
            nxt = pltpu.async_copy(table_hbm.at[idx_v.at[k + 1].at[0]],
                                   rows[b], gsems[b])
            gcp.wait()
            pltpu.sync_copy(rows[k % 2], acc_sh.at[idx_v.at[k].at[1]],
                            add=True)
            gcp = nxt
        gcp.wait()

        @pl.when(cmax - 1 < n_t)
        def _():
            pltpu.sync_copy(rows[(cmax - 1) % 2],
                            acc_sh.at[idx_v.at[cmax - 1].at[1]], add=True)

        plsc.subcore_barrier()
        pltpu.sync_copy(
            acc_sh.at[pl.ds(row0, ROWS_PER_TILE)],
            out_hbm.at[c].at[pl.ds(row0, ROWS_PER_TILE)],
        )

    return msg_kernel


def _mm1_body(x_ref, w_ref, o_ref):
    h = jnp.dot(x_ref[...], w_ref[...], preferred_element_type=jnp.float32)
    pad = o_ref.shape[0] - h.shape[0]
    o_ref[...] = jnp.concatenate(
        [h, jnp.zeros((pad, h.shape[1]), jnp.float32)], axis=0)


def _scale_body(dp_ref, h_ref, o_ref):
    dis = lax.rsqrt(dp_ref[0] + dp_ref[1] + 1.0)
    o_ref[...] = dis * h_ref[...]


def _layer_body(dp_ref, p_ref, hp_ref, w_ref, b_ref, o_ref):
    dis = lax.rsqrt(dp_ref[0] + dp_ref[1] + 1.0)
    z = dis * (p_ref[0] + p_ref[1] + hp_ref[...]) + b_ref[...]
    a = jnp.maximum(z, 0.0)
    h2 = jnp.dot(a, w_ref[...], preferred_element_type=jnp.float32)
    o_ref[...] = dis * h2


def _out_body(dp_ref, q_ref, hp_ref, w_ref, b_ref, bo_ref, o_ref):
    nf = o_ref.shape[0]
    dis = lax.rsqrt(dp_ref[0, :nf] + dp_ref[1, :nf] + 1.0)
    z = dis * (q_ref[0, :nf] + q_ref[1, :nf] + hp_ref[:nf]) + b_ref[...]
    o_ref[...] = jnp.dot(z, w_ref[...],
                         preferred_element_type=jnp.float32) + bo_ref[...]


def _full(shape):
    return pl.BlockSpec(shape, lambda: tuple(0 for _ in shape))


def _tc_call(body, in_shapes, out_shape):
    return pl.pallas_call(
        body,
        in_specs=[_full(s) for s in in_shapes],
        out_specs=_full(out_shape),
        out_shape=jax.ShapeDtypeStruct(out_shape, jnp.float32),
    )


def kernel(x, edge_index, W1, b1, W2, b2, Wout, bout):
    E = edge_index.shape[1]
    total_chunks = E // CHUNK
    assert total_chunks * CHUNK == E

    idx3 = edge_index.astype(jnp.int32).reshape(2, total_chunks,
                                                CHUNK).transpose(1, 0, 2)
    x_flat = x.reshape(NF, 8 * D)
    eye8 = jnp.eye(8, dtype=jnp.float32)
    W1b = jnp.kron(eye8, W1)
    W2b = jnp.kron(eye8, W2)
    Woutb = jnp.kron(eye8, Wout)
    b1f = jnp.tile(b1, 8)[None]
    b2f = jnp.tile(b2, 8)[None]
    boutf = jnp.tile(bout, 8)[None]
    zeros_hbm = jnp.zeros((ROWS_PER_TILE, H), jnp.float32)
    ones_hbm = jnp.ones((CHUNK, H), jnp.float32)

    NPF = NP * H // 128

    h1f = _tc_call(_mm1_body, [(NF, 8 * D), (8 * D, D)], (NPF, D))(x_flat,
                                                                   W1b)

    degp = _make_deg_kernel(total_chunks)(idx3, zeros_hbm, ones_hbm)
    degpf = degp.reshape(2, NPF, 128)

    h1pf = _tc_call(_scale_body, [(2, NPF, 128), (NPF, 128)],
                    (NPF, 128))(degpf, h1f)

    p = _make_msg_kernel(total_chunks)(h1pf.reshape(NP, H), idx3, zeros_hbm)

    h2pf = _tc_call(
        _layer_body,
        [(2, NPF, 128), (2, NPF, 128), (NPF, 128), (128, 128), (1, 128)],
        (NPF, 128))(degpf, p.reshape(2, NPF, 128), h1pf, W2b, b1f)

    q = _make_msg_kernel(total_chunks)(h2pf.reshape(NP, H), idx3, zeros_hbm)

    outf = _tc_call(
        _out_body,
        [(2, NPF, 128), (2, NPF, 128), (NPF, 128), (128, 8 * C), (1, 128),
         (1, 8 * C)],
        (NF, 8 * C))(degpf, q.reshape(2, NPF, 128), h2pf, Woutb, b2f, boutf)

    return outf.reshape(N, C)
